# baseline (speedup 1.0000x reference)
import functools

import jax
import jax.numpy as jnp
from jax import lax
from jax.experimental import pallas as pl
from jax.experimental.pallas import tpu as pltpu
from jax.experimental.pallas import tpu_sc as plsc

B = 16384
F = 26
V = 100001
D = 16
C = 13
T = F + 2

NC, NS = 2, 16
NW = NC * NS
BPW = B // NW
NB = 128
NCHUNK = BPW // NB
NBF = NB * F
KI = NBF // 128


def _cont_tc_body(x_ref, w_ref, b_ref, fte_ref, o_ref):
    acc = lax.dot_general(x_ref[...], w_ref[...], (((1,), (1,)), ((), ())),
                          preferred_element_type=jnp.float32)
    o_ref[...] = acc + b_ref[...] + fte_ref[F:F + 1, :]


def _cont_token(x_cont, cont_W, cont_b, fte):
    blk = 512
    return pl.pallas_call(
        _cont_tc_body,
        grid=(B // blk,),
        in_specs=[
            pl.BlockSpec((blk, C), lambda i: (i, 0)),
            pl.BlockSpec((D, C), lambda i: (0, 0)),
            pl.BlockSpec((1, D), lambda i: (0, 0)),
            pl.BlockSpec((F + 1, D), lambda i: (0, 0)),
        ],
        out_specs=pl.BlockSpec((blk, D), lambda i: (i, 0)),
        out_shape=jax.ShapeDtypeStruct((B, D), jnp.float32),
    )(x_cont, cont_W, cont_b.reshape(1, D), fte)


def _sc_body(table, x1d, off1d, fte, cls_row, cont_tok, out,
             xb, ib, offv, ftev, clsv, gb, cb, ob, sem):
    wid = lax.axis_index("s") * NC + lax.axis_index("c")
    wb = wid * BPW

    pltpu.sync_copy(off1d, offv)
    pltpu.sync_copy(fte, ftev)
    pltpu.sync_copy(cls_row, clsv)

    ftes = [ftev[f] for f in range(F)]
    cls_vec = clsv[0]

    def chunk_body(c, _):
        b0 = wb + c * NB

        pltpu.sync_copy(x1d.at[pl.ds(b0 * F, NBF)], xb)
        pltpu.sync_copy(cont_tok.at[pl.ds(b0, NB)], cb)

        def idx_body(j, _):
            for i in range(8):
                s = pl.ds(j * 128 + i * 16, 16)
                ib[j, pl.ds(i * 16, 16)] = xb[s] + offv[s]
            return 0
        lax.fori_loop(0, KI, idx_body, 0)

        copies = [
            pltpu.async_copy(table.at[ib.at[j]],
                             gb.at[pl.ds(j * 128, 128)], sem)
            for j in range(KI)
        ]
        for cp in copies:
            cp.wait()

        def asm_body(b, _):
            g0 = b * F
            o0 = b * T
            ob[o0] = cls_vec
            for f in range(F):
                ob[o0 + 1 + f] = gb[g0 + f] + ftes[f]
            ob[o0 + F + 1] = cb[b]
            return 0
        lax.fori_loop(0, NB, asm_body, 0)

        pltpu.sync_copy(ob, out.at[pl.ds(b0 * T, NB * T)])
        return 0

    lax.fori_loop(0, NCHUNK, chunk_body, 0)


def kernel(x_cat, x_cont, cat_emb, feature_type_embed, cont_W, cont_b,
           cls_token):
    table = cat_emb.reshape(F * V, D)
    x1d = x_cat.astype(jnp.int32).reshape(B * F)
    off1d = jnp.tile(jnp.arange(F, dtype=jnp.int32) * V, NB)
    cls_row = cls_token.reshape(1, D)
    cont_tok = _cont_token(x_cont, cont_W, cont_b, feature_type_embed)

    mesh = plsc.VectorSubcoreMesh(core_axis_name="c", subcore_axis_name="s")
    sc = pl.kernel(
        _sc_body,
        out_type=jax.ShapeDtypeStruct((B * T, D), jnp.float32),
        mesh=mesh,
        compiler_params=pltpu.CompilerParams(use_tc_tiling_on_sc=False),
        scratch_types=[
            pltpu.VMEM((NBF,), jnp.int32),
            pltpu.VMEM((KI, 128), jnp.int32),
            pltpu.VMEM((NBF,), jnp.int32),
            pltpu.VMEM((F + 1, D), jnp.float32),
            pltpu.VMEM((1, D), jnp.float32),
            pltpu.VMEM((NBF, D), jnp.float32),
            pltpu.VMEM((NB, D), jnp.float32),
            pltpu.VMEM((NB * T, D), jnp.float32),
            pltpu.SemaphoreType.DMA,
        ],
    )
    out = sc(table, x1d, off1d, feature_type_embed, cls_row, cont_tok)
    return out.reshape(B, T, D)
